# Initial kernel scaffold; baseline (speedup 1.0000x reference)
import jax
import jax.numpy as jnp
from jax.experimental import pallas as pl


def kernel(image, text, centroids):
    raise NotImplementedError("write your pallas kernel here")


# speedup vs baseline: 1.8398x; 1.8398x over previous
import functools

import jax
import jax.numpy as jnp
from jax import lax
from jax.experimental import pallas as pl
from jax.experimental.pallas import tpu as pltpu
from jax.experimental.pallas import tpu_sc as plsc

_M = 32
_K = 256
_D = 512
_d = _D // _M
_B = 4096
_NB = 2 * _B

_G = 4
_MG = _M // _G
_GD = _MG * _d
_GK = _MG * _K

_RV = 256
_RS = 256

_PREC = lax.Precision.DEFAULT

_NW = 32
_PW = (_NB * _M) // _NW
_CH = _PW // 128


def _encode_body(v_ref, wg_ref, idx_ref, loss_ref):
    v = v_ref[...]
    iota = lax.broadcasted_iota(jnp.int32, (_RV, _K), 1)
    loss = jnp.zeros((_RV, 1), jnp.float32)
    cols = []
    for g in range(_G):
        wgg = wg_ref[g]
        vg = v[:, g * _GD:(g + 1) * _GD]
        s = lax.dot(vg, wgg, precision=_PREC)
        cn = jnp.sum(wgg * wgg, axis=0, keepdims=True)
        vsq = vg * vg
        for mm in range(_MG):
            vn = jnp.sum(vsq[:, mm * _d:(mm + 1) * _d], axis=1, keepdims=True)
            dist = vn + cn[:, mm * _K:(mm + 1) * _K] - 2.0 * s[:, mm * _K:(mm + 1) * _K]
            minv = jnp.min(dist, axis=1, keepdims=True)
            first = jnp.min(jnp.where(dist == minv, iota, _K), axis=1, keepdims=True)
            cols.append(first + (g * _MG + mm) * _K)
            loss = loss + minv
    idx_ref[...] = jnp.concatenate(cols, axis=1)
    loss_ref[...] = loss


_encode = pl.pallas_call(
    _encode_body,
    grid=(_NB // _RV,),
    in_specs=[
        pl.BlockSpec((_RV, _D), lambda i: (i, 0)),
        pl.BlockSpec((_G, _GD, _GK), lambda i: (0, 0, 0)),
    ],
    out_specs=[
        pl.BlockSpec((_RV, _M), lambda i: (i, 0)),
        pl.BlockSpec((_RV, 1), lambda i: (i, 0)),
    ],
    out_shape=[
        jax.ShapeDtypeStruct((_NB, _M), jnp.int32),
        jax.ShapeDtypeStruct((_NB, 1), jnp.float32),
    ],
)


@functools.cache
def _make_gather():
    mesh = plsc.VectorSubcoreMesh(core_axis_name="c", subcore_axis_name="s")

    @functools.partial(
        pl.kernel,
        mesh=mesh,
        out_type=jax.ShapeDtypeStruct((_NB * _M, _d), jnp.float32),
        scratch_types=[
            pltpu.VMEM((_CH, 128), jnp.int32),
            pltpu.VMEM((128, _d), jnp.float32),
            pltpu.SemaphoreType.DMA,
        ],
        compiler_params=pltpu.CompilerParams(use_tc_tiling_on_sc=False),
    )
    def gather(table_hbm, idx_hbm, out_hbm, idx_v, rows_v, sem):
        wid = lax.axis_index("s") * 2 + lax.axis_index("c")
        base = wid * _PW
        pltpu.sync_copy(idx_hbm.at[wid], idx_v)

        def chunk(j, carry):
            pltpu.async_copy(table_hbm.at[idx_v.at[j]], rows_v, sem).wait()
            pltpu.sync_copy(rows_v, out_hbm.at[pl.ds(base + j * 128, 128)])
            return carry

        lax.fori_loop(0, _CH, chunk, 0)

    return gather


def _sim_body(img_ref, txtT_ref, out_ref):
    logits = 100.0 * lax.dot(img_ref[...], txtT_ref[...], precision=_PREC)
    mx = jnp.max(logits, axis=1, keepdims=True)
    e = jnp.exp(logits - mx)
    out_ref[...] = e / jnp.sum(e, axis=1, keepdims=True)


_sim = pl.pallas_call(
    _sim_body,
    grid=(_B // _RS,),
    in_specs=[
        pl.BlockSpec((_RS, _D), lambda i: (i, 0)),
        pl.BlockSpec((_D, _B), lambda i: (0, 0)),
    ],
    out_specs=pl.BlockSpec((_RS, _B), lambda i: (i, 0)),
    out_shape=jax.ShapeDtypeStruct((_B, _B), jnp.float32),
)


def kernel(image, text, centroids):
    vecs = jnp.concatenate([image, text], axis=0)
    cT = centroids.reshape(_G, _MG, _K, _d).transpose(0, 1, 3, 2)
    eye = jnp.eye(_MG, dtype=jnp.float32)
    wg = (eye[None, :, None, :, None] * cT[:, :, :, None, :]).reshape(_G, _GD, _GK)

    idx, row_loss = _encode(vecs, wg)
    quant_loss = (2.0 / _B) * jnp.sum(row_loss)

    q3 = idx.reshape(_NW, _CH, 128)
    rows = _make_gather()(centroids.reshape(_M * _K, _d), q3)
    qv = rows.reshape(_NB, _D)

    similarity = _sim(qv[:_B], qv[_B:].T)
    return similarity, quant_loss


# speedup vs baseline: 3.0428x; 1.6538x over previous
import functools

import jax
import jax.numpy as jnp
from jax import lax
from jax.experimental import pallas as pl
from jax.experimental.pallas import tpu as pltpu
from jax.experimental.pallas import tpu_sc as plsc

_M = 32
_K = 256
_D = 512
_d = _D // _M
_B = 4096
_NB = 2 * _B

_G = 4
_MG = _M // _G
_GD = _MG * _d
_GK = _MG * _K

_RV = 256
_RC = 256
_RS = 256

_PREC = lax.Precision.DEFAULT

_NW = 32
_PW = (_NB * _M) // _NW
_CH = _PW // 128


def _encode_body(v_ref, wg_ref, cn_ref, idx_ref, loss_ref):
    v = v_ref[...]
    ss = [lax.dot(v[:, g * _GD:(g + 1) * _GD], wg_ref[g], precision=_PREC)
          for g in range(_G)]
    vsq = v * v
    iota_f = lax.broadcasted_iota(jnp.int32, (_RC, _K), 1).astype(jnp.float32)
    for rc in range(_RV // _RC):
        r0 = rc * _RC
        loss_c = None
        idx_c = []
        for m in range(_M):
            g, mm = divmod(m, _MG)
            sm = ss[g][r0:r0 + _RC, mm * _K:(mm + 1) * _K]
            vn = jnp.sum(vsq[r0:r0 + _RC, m * _d:(m + 1) * _d], axis=1, keepdims=True)
            dist = (vn + cn_ref[:, m * _K:(m + 1) * _K]) - 2.0 * sm
            minv = jnp.min(dist, axis=1, keepdims=True)
            first = jnp.min(jnp.where(dist == minv, iota_f, float(_K)),
                            axis=1, keepdims=True)
            idx_c.append(first.astype(jnp.int32) + m * _K)
            loss_c = minv if loss_c is None else loss_c + minv
        idx_ref[r0:r0 + _RC, :] = jnp.concatenate(idx_c, axis=1)
        loss_ref[r0:r0 + _RC, :] = loss_c


_encode = pl.pallas_call(
    _encode_body,
    grid=(_NB // _RV,),
    in_specs=[
        pl.BlockSpec((_RV, _D), lambda i: (i, 0)),
        pl.BlockSpec((_G, _GD, _GK), lambda i: (0, 0, 0)),
        pl.BlockSpec((1, _M * _K), lambda i: (0, 0)),
    ],
    out_specs=[
        pl.BlockSpec((_RV, _M), lambda i: (i, 0)),
        pl.BlockSpec((_RV, 1), lambda i: (i, 0)),
    ],
    out_shape=[
        jax.ShapeDtypeStruct((_NB, _M), jnp.int32),
        jax.ShapeDtypeStruct((_NB, 1), jnp.float32),
    ],
)


@functools.cache
def _make_gather():
    mesh = plsc.VectorSubcoreMesh(core_axis_name="c", subcore_axis_name="s")

    @functools.partial(
        pl.kernel,
        mesh=mesh,
        out_type=jax.ShapeDtypeStruct((_NB * _M, _d), jnp.float32),
        scratch_types=[
            pltpu.VMEM((_CH, 128), jnp.int32),
            pltpu.VMEM((128, _d), jnp.float32),
            pltpu.SemaphoreType.DMA,
        ],
        compiler_params=pltpu.CompilerParams(use_tc_tiling_on_sc=False),
    )
    def gather(table_hbm, idx_hbm, out_hbm, idx_v, rows_v, sem):
        wid = lax.axis_index("s") * 2 + lax.axis_index("c")
        base = wid * _PW
        pltpu.sync_copy(idx_hbm.at[wid], idx_v)

        def chunk(j, carry):
            pltpu.async_copy(table_hbm.at[idx_v.at[j]], rows_v, sem).wait()
            pltpu.sync_copy(rows_v, out_hbm.at[pl.ds(base + j * 128, 128)])
            return carry

        lax.fori_loop(0, _CH, chunk, 0)

    return gather


def _sim_body(img_ref, txt_ref, out_ref):
    logits = 100.0 * lax.dot_general(
        img_ref[...], txt_ref[...], (((1,), (1,)), ((), ())), precision=_PREC)
    mx = jnp.max(logits, axis=1, keepdims=True)
    e = jnp.exp(logits - mx)
    out_ref[...] = e / jnp.sum(e, axis=1, keepdims=True)


_sim = pl.pallas_call(
    _sim_body,
    grid=(_B // _RS,),
    in_specs=[
        pl.BlockSpec((_RS, _D), lambda i: (i, 0)),
        pl.BlockSpec((_B, _D), lambda i: (1, 0)),
    ],
    out_specs=pl.BlockSpec((_RS, _B), lambda i: (i, 0)),
    out_shape=jax.ShapeDtypeStruct((_B, _B), jnp.float32),
)


def kernel(image, text, centroids):
    vecs = jnp.concatenate([image, text], axis=0)
    cT = centroids.reshape(_G, _MG, _K, _d).transpose(0, 1, 3, 2)
    eye = jnp.eye(_MG, dtype=jnp.float32)
    wg = (eye[None, :, None, :, None] * cT[:, :, :, None, :]).reshape(_G, _GD, _GK)
    cn = jnp.sum(centroids ** 2, axis=2).reshape(1, _M * _K)

    idx, row_loss = _encode(vecs, wg, cn)
    quant_loss = (2.0 / _B) * jnp.sum(row_loss)

    q3 = idx.reshape(_NW, _CH, 128)
    rows = _make_gather()(centroids.reshape(_M * _K, _d), q3)
    qv = rows.reshape(_NB, _D)

    similarity = _sim(qv, qv)
    return similarity, quant_loss


# speedup vs baseline: 3.3525x; 1.1018x over previous
import functools

import jax
import jax.numpy as jnp
from jax import lax
from jax.experimental import pallas as pl
from jax.experimental.pallas import tpu as pltpu
from jax.experimental.pallas import tpu_sc as plsc

_M = 32
_K = 256
_D = 512
_d = _D // _M
_B = 4096
_NB = 2 * _B

_G = 4
_MG = _M // _G
_GD = _MG * _d
_GK = _MG * _K

_RV = 256
_RC = 256
_RS = 256

_PREC = lax.Precision.DEFAULT

_NW = 32
_PW = (_NB * _M) // _NW
_CH = _PW // 128
_FK = 16


def _encode_body(v_ref, wg_ref, cn_ref, idx_ref, loss_ref):
    v = v_ref[...]
    ss = [lax.dot(v[:, g * _GD:(g + 1) * _GD], wg_ref[g], precision=_PREC)
          for g in range(_G)]
    vsq = v * v
    iota_f = lax.broadcasted_iota(jnp.int32, (_RC, _K), 1).astype(jnp.float32)
    for rc in range(_RV // _RC):
        r0 = rc * _RC
        loss_c = None
        idx_c = []
        for m in range(_M):
            g, mm = divmod(m, _MG)
            sm = ss[g][r0:r0 + _RC, mm * _K:(mm + 1) * _K]
            vn = jnp.sum(vsq[r0:r0 + _RC, m * _d:(m + 1) * _d], axis=1, keepdims=True)
            dist = (vn + cn_ref[:, m * _K:(m + 1) * _K]) - 2.0 * sm
            minv = jnp.min(dist, axis=1, keepdims=True)
            first = jnp.min(jnp.where(dist == minv, iota_f, float(_K)),
                            axis=1, keepdims=True)
            idx_c.append(first.astype(jnp.int32) + m * _K)
            loss_c = minv if loss_c is None else loss_c + minv
        idx_ref[r0:r0 + _RC, :] = jnp.concatenate(idx_c, axis=1)
        loss_ref[r0:r0 + _RC, :] = loss_c


_encode = pl.pallas_call(
    _encode_body,
    grid=(_NB // _RV,),
    in_specs=[
        pl.BlockSpec((_RV, _D), lambda i: (i, 0)),
        pl.BlockSpec((_G, _GD, _GK), lambda i: (0, 0, 0)),
        pl.BlockSpec((1, _M * _K), lambda i: (0, 0)),
    ],
    out_specs=[
        pl.BlockSpec((_RV, _M), lambda i: (i, 0)),
        pl.BlockSpec((_RV, 1), lambda i: (i, 0)),
    ],
    out_shape=[
        jax.ShapeDtypeStruct((_NB, _M), jnp.int32),
        jax.ShapeDtypeStruct((_NB, 1), jnp.float32),
    ],
)


@functools.cache
def _make_gather():
    mesh = plsc.VectorSubcoreMesh(core_axis_name="c", subcore_axis_name="s")

    @functools.partial(
        pl.kernel,
        mesh=mesh,
        out_type=jax.ShapeDtypeStruct((_NB * _M, _d), jnp.float32),
        scratch_types=[
            pltpu.VMEM((_CH, 128), jnp.int32),
            pltpu.VMEM((_FK * 128, _d), jnp.float32),
            pltpu.SemaphoreType.DMA,
        ],
        compiler_params=pltpu.CompilerParams(use_tc_tiling_on_sc=False),
    )
    def gather(table_hbm, idx_hbm, out_hbm, idx_v, rows_v, sem):
        wid = lax.axis_index("s") * 2 + lax.axis_index("c")
        base = wid * _PW
        pltpu.sync_copy(idx_hbm.at[wid], idx_v)

        def group(jj, carry):
            copies = [
                pltpu.async_copy(
                    table_hbm.at[idx_v.at[jj * _FK + t]],
                    rows_v.at[pl.ds(t * 128, 128)], sem)
                for t in range(_FK)
            ]
            for c in copies:
                c.wait()
            pltpu.sync_copy(rows_v, out_hbm.at[pl.ds(base + jj * _FK * 128, _FK * 128)])
            return carry

        lax.fori_loop(0, _CH // _FK, group, 0)

    return gather


def _sim_body(img_ref, txt_ref, out_ref):
    logits = 100.0 * lax.dot_general(
        img_ref[...], txt_ref[...], (((1,), (1,)), ((), ())), precision=_PREC)
    mx = jnp.max(logits, axis=1, keepdims=True)
    e = jnp.exp(logits - mx)
    out_ref[...] = e / jnp.sum(e, axis=1, keepdims=True)


_sim = pl.pallas_call(
    _sim_body,
    grid=(_B // _RS,),
    in_specs=[
        pl.BlockSpec((_RS, _D), lambda i: (i, 0)),
        pl.BlockSpec((_B, _D), lambda i: (1, 0)),
    ],
    out_specs=pl.BlockSpec((_RS, _B), lambda i: (i, 0)),
    out_shape=jax.ShapeDtypeStruct((_B, _B), jnp.float32),
)


def kernel(image, text, centroids):
    vecs = jnp.concatenate([image, text], axis=0)
    cT = centroids.reshape(_G, _MG, _K, _d).transpose(0, 1, 3, 2)
    eye = jnp.eye(_MG, dtype=jnp.float32)
    wg = (eye[None, :, None, :, None] * cT[:, :, :, None, :]).reshape(_G, _GD, _GK)
    cn = jnp.sum(centroids ** 2, axis=2).reshape(1, _M * _K)

    idx, row_loss = _encode(vecs, wg, cn)
    quant_loss = (2.0 / _B) * jnp.sum(row_loss)

    q3 = idx.reshape(_NW, _CH, 128)
    rows = _make_gather()(centroids.reshape(_M * _K, _d), q3)
    qv = rows.reshape(_NB, _D)

    similarity = _sim(qv, qv)
    return similarity, quant_loss


# speedup vs baseline: 5.8858x; 1.7556x over previous
import functools

import jax
import jax.numpy as jnp
from jax import lax
from jax.experimental import pallas as pl
from jax.experimental.pallas import tpu as pltpu
from jax.experimental.pallas import tpu_sc as plsc

_M = 32
_K = 256
_D = 512
_d = _D // _M
_B = 4096
_NB = 2 * _B

_G = 4
_MG = _M // _G
_GD = _MG * _d
_GK = _MG * _K

_RT = 512
_RS = 256

_PREC = lax.Precision.DEFAULT

_NW = 32
_PW = (_NB * _M) // _NW
_CH = _PW // 128
_FK = 16


def _encode_body(vT_ref, wgT_ref, cnT_ref, idxT_ref, lossT_ref):
    vT = vT_ref[...]
    ss = [lax.dot(wgT_ref[g], vT[g * _GD:(g + 1) * _GD, :], precision=_PREC)
          for g in range(_G)]
    vsqT = vT * vT
    iota_col = lax.broadcasted_iota(jnp.int32, (_K, 1), 0).astype(jnp.float32)
    idx_rows = []
    loss_acc = None
    for m in range(_M):
        g, mm = divmod(m, _MG)
        smT = ss[g][mm * _K:(mm + 1) * _K, :]
        vnT = jnp.sum(vsqT[m * _d:(m + 1) * _d, :], axis=0, keepdims=True)
        distT = (vnT + cnT_ref[m * _K:(m + 1) * _K, :]) - 2.0 * smT
        minvT = jnp.min(distT, axis=0, keepdims=True)
        firstT = jnp.min(jnp.where(distT == minvT, iota_col, float(_K)),
                         axis=0, keepdims=True)
        idx_rows.append(firstT.astype(jnp.int32) + m * _K)
        loss_acc = minvT if loss_acc is None else loss_acc + minvT
    idxT_ref[...] = jnp.concatenate(idx_rows, axis=0)
    lossT_ref[...] = loss_acc


_encode = pl.pallas_call(
    _encode_body,
    grid=(_NB // _RT,),
    in_specs=[
        pl.BlockSpec((_D, _RT), lambda i: (0, i)),
        pl.BlockSpec((_G, _GK, _GD), lambda i: (0, 0, 0)),
        pl.BlockSpec((_M * _K, 1), lambda i: (0, 0)),
    ],
    out_specs=[
        pl.BlockSpec((_M, _RT), lambda i: (0, i)),
        pl.BlockSpec((1, _RT), lambda i: (0, i)),
    ],
    out_shape=[
        jax.ShapeDtypeStruct((_M, _NB), jnp.int32),
        jax.ShapeDtypeStruct((1, _NB), jnp.float32),
    ],
)


@functools.cache
def _make_gather():
    mesh = plsc.VectorSubcoreMesh(core_axis_name="c", subcore_axis_name="s")

    @functools.partial(
        pl.kernel,
        mesh=mesh,
        out_type=jax.ShapeDtypeStruct((_NB * _M, _d), jnp.float32),
        scratch_types=[
            pltpu.VMEM((_CH, 128), jnp.int32),
            pltpu.VMEM((_FK * 128, _d), jnp.float32),
            pltpu.SemaphoreType.DMA,
        ],
        compiler_params=pltpu.CompilerParams(use_tc_tiling_on_sc=False),
    )
    def gather(table_hbm, idx_hbm, out_hbm, idx_v, rows_v, sem):
        wid = lax.axis_index("s") * 2 + lax.axis_index("c")
        base = wid * _PW
        pltpu.sync_copy(idx_hbm.at[wid], idx_v)

        def group(jj, carry):
            copies = [
                pltpu.async_copy(
                    table_hbm.at[idx_v.at[jj * _FK + t]],
                    rows_v.at[pl.ds(t * 128, 128)], sem)
                for t in range(_FK)
            ]
            for c in copies:
                c.wait()
            pltpu.sync_copy(rows_v, out_hbm.at[pl.ds(base + jj * _FK * 128, _FK * 128)])
            return carry

        lax.fori_loop(0, _CH // _FK, group, 0)

    return gather


def _sim_body(img_ref, txt_ref, out_ref):
    logits = 100.0 * lax.dot_general(
        img_ref[...], txt_ref[...], (((1,), (1,)), ((), ())), precision=_PREC)
    mx = jnp.max(logits, axis=1, keepdims=True)
    e = jnp.exp(logits - mx)
    out_ref[...] = e / jnp.sum(e, axis=1, keepdims=True)


_sim = pl.pallas_call(
    _sim_body,
    grid=(_B // _RS,),
    in_specs=[
        pl.BlockSpec((_RS, _D), lambda i: (i, 0)),
        pl.BlockSpec((_B, _D), lambda i: (1, 0)),
    ],
    out_specs=pl.BlockSpec((_RS, _B), lambda i: (i, 0)),
    out_shape=jax.ShapeDtypeStruct((_B, _B), jnp.float32),
)


def kernel(image, text, centroids):
    vecsT = jnp.concatenate([image, text], axis=0).T
    cg = centroids.reshape(_G, _MG, _K, _d).transpose(0, 2, 1, 3)
    eye = jnp.eye(_MG, dtype=jnp.float32)
    wgT = (eye[None, :, None, :, None] * cg[:, None, :, :, :]).reshape(_G, _GK, _GD)
    cnT = jnp.sum(centroids ** 2, axis=2).reshape(_M * _K, 1)

    idxT, lossT = _encode(vecsT, wgT, cnT)
    quant_loss = (2.0 / _B) * jnp.sum(lossT)

    q3 = idxT.T.reshape(_NW, _CH, 128)
    rows = _make_gather()(centroids.reshape(_M * _K, _d), q3)
    qv = rows.reshape(_NB, _D)

    similarity = _sim(qv, qv)
    return similarity, quant_loss


# speedup vs baseline: 6.1102x; 1.0381x over previous
import functools

import jax
import jax.numpy as jnp
from jax import lax
from jax.experimental import pallas as pl
from jax.experimental.pallas import tpu as pltpu
from jax.experimental.pallas import tpu_sc as plsc

_M = 32
_K = 256
_D = 512
_d = _D // _M
_B = 4096
_NB = 2 * _B

_G = 4
_MG = _M // _G
_GD = _MG * _d
_GK = _MG * _K

_RT = 1024
_RS = 512

_PREC = lax.Precision.DEFAULT

_NW = 32
_PW = (_NB * _M) // _NW
_CH = _PW // 128
_FK = 16


def _encode_body(vT_ref, wgT_ref, cnT_ref, idxT_ref, lossT_ref):
    vT = vT_ref[...]
    ss = [lax.dot(wgT_ref[g], vT[g * _GD:(g + 1) * _GD, :], precision=_PREC)
          for g in range(_G)]
    vsqT = vT * vT
    iota_col = lax.broadcasted_iota(jnp.int32, (_K, 1), 0).astype(jnp.float32)
    idx_rows = []
    loss_acc = None
    for m in range(_M):
        g, mm = divmod(m, _MG)
        smT = ss[g][mm * _K:(mm + 1) * _K, :]
        vnT = jnp.sum(vsqT[m * _d:(m + 1) * _d, :], axis=0, keepdims=True)
        distT = (vnT + cnT_ref[m * _K:(m + 1) * _K, :]) - smT
        minvT = jnp.min(distT, axis=0, keepdims=True)
        firstT = jnp.min(jnp.where(distT == minvT, iota_col, float(_K)),
                         axis=0, keepdims=True)
        idx_rows.append(firstT.astype(jnp.int32) + m * _K)
        loss_acc = minvT if loss_acc is None else loss_acc + minvT
    idxT_ref[...] = jnp.concatenate(idx_rows, axis=0)
    lossT_ref[...] = loss_acc


_encode = pl.pallas_call(
    _encode_body,
    grid=(_NB // _RT,),
    in_specs=[
        pl.BlockSpec((_D, _RT), lambda i: (0, i)),
        pl.BlockSpec((_G, _GK, _GD), lambda i: (0, 0, 0)),
        pl.BlockSpec((_M * _K, 1), lambda i: (0, 0)),
    ],
    out_specs=[
        pl.BlockSpec((_M, _RT), lambda i: (0, i)),
        pl.BlockSpec((1, _RT), lambda i: (0, i)),
    ],
    out_shape=[
        jax.ShapeDtypeStruct((_M, _NB), jnp.int32),
        jax.ShapeDtypeStruct((1, _NB), jnp.float32),
    ],
)


@functools.cache
def _make_gather():
    mesh = plsc.VectorSubcoreMesh(core_axis_name="c", subcore_axis_name="s")

    @functools.partial(
        pl.kernel,
        mesh=mesh,
        out_type=jax.ShapeDtypeStruct((_NB * _M, _d), jnp.float32),
        scratch_types=[
            pltpu.VMEM((_CH, 128), jnp.int32),
            pltpu.VMEM((_FK * 128, _d), jnp.float32),
            pltpu.SemaphoreType.DMA,
        ],
        compiler_params=pltpu.CompilerParams(use_tc_tiling_on_sc=False),
    )
    def gather(table_hbm, idx_hbm, out_hbm, idx_v, rows_v, sem):
        wid = lax.axis_index("s") * 2 + lax.axis_index("c")
        base = wid * _PW
        pltpu.sync_copy(idx_hbm.at[wid], idx_v)

        def group(jj, carry):
            copies = [
                pltpu.async_copy(
                    table_hbm.at[idx_v.at[jj * _FK + t]],
                    rows_v.at[pl.ds(t * 128, 128)], sem)
                for t in range(_FK)
            ]
            for c in copies:
                c.wait()
            pltpu.sync_copy(rows_v, out_hbm.at[pl.ds(base + jj * _FK * 128, _FK * 128)])
            return carry

        lax.fori_loop(0, _CH // _FK, group, 0)

    return gather


def _sim_body(img_ref, txt_ref, out_ref):
    logits = 100.0 * lax.dot_general(
        img_ref[...], txt_ref[...], (((1,), (1,)), ((), ())), precision=_PREC)
    mx = jnp.max(logits, axis=1, keepdims=True)
    e = jnp.exp(logits - mx)
    out_ref[...] = e / jnp.sum(e, axis=1, keepdims=True)


_sim = pl.pallas_call(
    _sim_body,
    grid=(_B // _RS,),
    in_specs=[
        pl.BlockSpec((_RS, _D), lambda i: (i, 0)),
        pl.BlockSpec((_B, _D), lambda i: (1, 0)),
    ],
    out_specs=pl.BlockSpec((_RS, _B), lambda i: (i, 0)),
    out_shape=jax.ShapeDtypeStruct((_B, _B), jnp.float32),
)


def kernel(image, text, centroids):
    vecsT = jnp.concatenate([image, text], axis=0).T
    cg = centroids.reshape(_G, _MG, _K, _d).transpose(0, 2, 1, 3)
    eye = jnp.eye(_MG, dtype=jnp.float32)
    wgT = (2.0 * eye[None, :, None, :, None] * cg[:, None, :, :, :]).reshape(_G, _GK, _GD)
    cnT = jnp.sum(centroids ** 2, axis=2).reshape(_M * _K, 1)

    idxT, lossT = _encode(vecsT, wgT, cnT)
    quant_loss = (2.0 / _B) * jnp.sum(lossT)

    q3 = idxT.T.reshape(_NW, _CH, 128)
    rows = _make_gather()(centroids.reshape(_M * _K, _d), q3)
    qv = rows.reshape(_NB, _D)

    similarity = _sim(qv, qv)
    return similarity, quant_loss


# speedup vs baseline: 6.5095x; 1.0653x over previous
import functools

import jax
import jax.numpy as jnp
from jax import lax
from jax.experimental import pallas as pl
from jax.experimental.pallas import tpu as pltpu
from jax.experimental.pallas import tpu_sc as plsc

_M = 32
_K = 256
_D = 512
_d = _D // _M
_B = 4096
_NB = 2 * _B

_G = 4
_MG = _M // _G
_GD = _MG * _d
_GK = _MG * _K

_RT = 1024
_RS = 512

_PREC = lax.Precision.DEFAULT

_NW = 32
_PW = (_NB * _M) // _NW
_CH = _PW // 128
_FK = 16


def _encode_body(v_ref, wgT_ref, cnT_ref, idxT_ref, lossT_ref):
    vT = jnp.transpose(v_ref[...])
    ss = [lax.dot(wgT_ref[g], vT[g * _GD:(g + 1) * _GD, :], precision=_PREC)
          for g in range(_G)]
    vsqT = vT * vT
    iota_col = lax.broadcasted_iota(jnp.int32, (_K, 1), 0).astype(jnp.float32)
    idx_rows = []
    loss_acc = None
    for m in range(_M):
        g, mm = divmod(m, _MG)
        smT = ss[g][mm * _K:(mm + 1) * _K, :]
        vnT = jnp.sum(vsqT[m * _d:(m + 1) * _d, :], axis=0, keepdims=True)
        distT = (vnT + cnT_ref[m * _K:(m + 1) * _K, :]) - smT
        minvT = jnp.min(distT, axis=0, keepdims=True)
        firstT = jnp.min(jnp.where(distT == minvT, iota_col, float(_K)),
                         axis=0, keepdims=True)
        idx_rows.append(firstT.astype(jnp.int32) + m * _K)
        loss_acc = minvT if loss_acc is None else loss_acc + minvT
    idxT_ref[...] = jnp.concatenate(idx_rows, axis=0)
    lossT_ref[...] = loss_acc


_encode = pl.pallas_call(
    _encode_body,
    grid=(_B // _RT,),
    in_specs=[
        pl.BlockSpec((_RT, _D), lambda i: (i, 0)),
        pl.BlockSpec((_G, _GK, _GD), lambda i: (0, 0, 0)),
        pl.BlockSpec((_M * _K, 1), lambda i: (0, 0)),
    ],
    out_specs=[
        pl.BlockSpec((_M, _RT), lambda i: (0, i)),
        pl.BlockSpec((1, _RT), lambda i: (0, i)),
    ],
    out_shape=[
        jax.ShapeDtypeStruct((_M, _B), jnp.int32),
        jax.ShapeDtypeStruct((1, _B), jnp.float32),
    ],
)


@functools.cache
def _make_gather():
    mesh = plsc.VectorSubcoreMesh(core_axis_name="c", subcore_axis_name="s")

    @functools.partial(
        pl.kernel,
        mesh=mesh,
        out_type=jax.ShapeDtypeStruct((_NB * _M, _d), jnp.float32),
        scratch_types=[
            pltpu.VMEM((_CH, 128), jnp.int32),
            pltpu.VMEM((_FK * 128, _d), jnp.float32),
            pltpu.SemaphoreType.DMA,
        ],
        compiler_params=pltpu.CompilerParams(use_tc_tiling_on_sc=False),
    )
    def gather(table_hbm, idx_hbm, out_hbm, idx_v, rows_v, sem):
        wid = lax.axis_index("s") * 2 + lax.axis_index("c")
        base = wid * _PW
        pltpu.sync_copy(idx_hbm.at[wid], idx_v)

        def group(jj, carry):
            copies = [
                pltpu.async_copy(
                    table_hbm.at[idx_v.at[jj * _FK + t]],
                    rows_v.at[pl.ds(t * 128, 128)], sem)
                for t in range(_FK)
            ]
            for c in copies:
                c.wait()
            pltpu.sync_copy(rows_v, out_hbm.at[pl.ds(base + jj * _FK * 128, _FK * 128)])
            return carry

        lax.fori_loop(0, _CH // _FK, group, 0)

    return gather


def _sim_body(img_ref, txt_ref, out_ref):
    logits = 100.0 * lax.dot_general(
        img_ref[...], txt_ref[...], (((1,), (1,)), ((), ())), precision=_PREC)
    mx = jnp.max(logits, axis=1, keepdims=True)
    e = jnp.exp(logits - mx)
    out_ref[...] = e / jnp.sum(e, axis=1, keepdims=True)


_sim = pl.pallas_call(
    _sim_body,
    grid=(_B // _RS,),
    in_specs=[
        pl.BlockSpec((_RS, _D), lambda i: (i, 0)),
        pl.BlockSpec((_B, _D), lambda i: (1, 0)),
    ],
    out_specs=pl.BlockSpec((_RS, _B), lambda i: (i, 0)),
    out_shape=jax.ShapeDtypeStruct((_B, _B), jnp.float32),
)


def kernel(image, text, centroids):
    cg = centroids.reshape(_G, _MG, _K, _d).transpose(0, 2, 1, 3)
    eye = jnp.eye(_MG, dtype=jnp.float32)
    wgT = (2.0 * eye[None, :, None, :, None] * cg[:, None, :, :, :]).reshape(_G, _GK, _GD)
    cnT = jnp.sum(centroids ** 2, axis=2).reshape(_M * _K, 1)

    idxT_i, lossT_i = _encode(image, wgT, cnT)
    idxT_t, lossT_t = _encode(text, wgT, cnT)
    quant_loss = (2.0 / _B) * (jnp.sum(lossT_i) + jnp.sum(lossT_t))

    q3 = jnp.concatenate([idxT_i.T, idxT_t.T], axis=0).reshape(_NW, _CH, 128)
    rows = _make_gather()(centroids.reshape(_M * _K, _d), q3)
    qv = rows.reshape(_NB, _D)

    similarity = _sim(qv, qv)
    return similarity, quant_loss


# speedup vs baseline: 6.5686x; 1.0091x over previous
import functools

import jax
import jax.numpy as jnp
from jax import lax
from jax.experimental import pallas as pl
from jax.experimental.pallas import tpu as pltpu
from jax.experimental.pallas import tpu_sc as plsc

_M = 32
_K = 256
_D = 512
_d = _D // _M
_B = 4096
_NB = 2 * _B

_G = 4
_MG = _M // _G
_GD = _MG * _d
_GK = _MG * _K

_RT = 1024
_RS = 512

_PREC = lax.Precision.DEFAULT

_NW = 32
_PW = (_NB * _M) // _NW
_CH = _PW // 128
_FK = 16


def _encode_body(v_ref, wgT_ref, cnT_ref, idxT_ref, lossT_ref):
    vT = jnp.transpose(v_ref[...])
    ss = [lax.dot(wgT_ref[g], vT[g * _GD:(g + 1) * _GD, :], precision=_PREC)
          for g in range(_G)]
    vsqT = vT * vT
    iota_col = lax.broadcasted_iota(jnp.int32, (_K, 1), 0).astype(jnp.float32)
    idx_rows = []
    loss_acc = None
    for m in range(_M):
        g, mm = divmod(m, _MG)
        smT = ss[g][mm * _K:(mm + 1) * _K, :]
        vnT = jnp.sum(vsqT[m * _d:(m + 1) * _d, :], axis=0, keepdims=True)
        distT = (vnT + cnT_ref[m * _K:(m + 1) * _K, :]) - smT
        minvT = jnp.min(distT, axis=0, keepdims=True)
        firstT = jnp.min(jnp.where(distT == minvT, iota_col, float(_K)),
                         axis=0, keepdims=True)
        idx_rows.append(firstT.astype(jnp.int32) + m * _K)
        loss_acc = minvT if loss_acc is None else loss_acc + minvT
    idxT_ref[...] = jnp.concatenate(idx_rows, axis=0)
    lossT_ref[...] = loss_acc


_encode = pl.pallas_call(
    _encode_body,
    grid=(_B // _RT,),
    in_specs=[
        pl.BlockSpec((_RT, _D), lambda i: (i, 0)),
        pl.BlockSpec((_G, _GK, _GD), lambda i: (0, 0, 0)),
        pl.BlockSpec((_M * _K, 1), lambda i: (0, 0)),
    ],
    out_specs=[
        pl.BlockSpec((_M, _RT), lambda i: (0, i)),
        pl.BlockSpec((1, _RT), lambda i: (0, i)),
    ],
    out_shape=[
        jax.ShapeDtypeStruct((_M, _B), jnp.int32),
        jax.ShapeDtypeStruct((1, _B), jnp.float32),
    ],
)


@functools.cache
def _make_gather(nrows):
    mesh = plsc.VectorSubcoreMesh(core_axis_name="c", subcore_axis_name="s")
    pw = nrows // _NW
    ch = pw // 128

    @functools.partial(
        pl.kernel,
        mesh=mesh,
        out_type=jax.ShapeDtypeStruct((nrows, _d), jnp.float32),
        scratch_types=[
            pltpu.VMEM((ch, 128), jnp.int32),
            pltpu.VMEM((_FK * 128, _d), jnp.float32),
            pltpu.SemaphoreType.DMA,
        ],
        compiler_params=pltpu.CompilerParams(use_tc_tiling_on_sc=False),
    )
    def gather(table_hbm, idx_hbm, out_hbm, idx_v, rows_v, sem):
        wid = lax.axis_index("s") * 2 + lax.axis_index("c")
        base = wid * pw
        pltpu.sync_copy(idx_hbm.at[wid], idx_v)

        def group(jj, carry):
            copies = [
                pltpu.async_copy(
                    table_hbm.at[idx_v.at[jj * _FK + t]],
                    rows_v.at[pl.ds(t * 128, 128)], sem)
                for t in range(_FK)
            ]
            for c in copies:
                c.wait()
            pltpu.sync_copy(rows_v, out_hbm.at[pl.ds(base + jj * _FK * 128, _FK * 128)])
            return carry

        lax.fori_loop(0, ch // _FK, group, 0)

    return gather


def _sim_body(img_ref, txt_ref, out_ref):
    logits = 100.0 * lax.dot_general(
        img_ref[...], txt_ref[...], (((1,), (1,)), ((), ())), precision=_PREC)
    mx = jnp.max(logits, axis=1, keepdims=True)
    e = jnp.exp(logits - mx)
    out_ref[...] = e / jnp.sum(e, axis=1, keepdims=True)


_sim = pl.pallas_call(
    _sim_body,
    grid=(_B // _RS,),
    in_specs=[
        pl.BlockSpec((_RS, _D), lambda i: (i, 0)),
        pl.BlockSpec((_B, _D), lambda i: (0, 0)),
    ],
    out_specs=pl.BlockSpec((_RS, _B), lambda i: (i, 0)),
    out_shape=jax.ShapeDtypeStruct((_B, _B), jnp.float32),
)


def kernel(image, text, centroids):
    cg = centroids.reshape(_G, _MG, _K, _d).transpose(0, 2, 1, 3)
    eye = jnp.eye(_MG, dtype=jnp.float32)
    wgT = (2.0 * eye[None, :, None, :, None] * cg[:, None, :, :, :]).reshape(_G, _GK, _GD)
    cnT = jnp.sum(centroids ** 2, axis=2).reshape(_M * _K, 1)

    table = centroids.reshape(_M * _K, _d)
    gat = _make_gather(_B * _M)

    idxT_i, lossT_i = _encode(image, wgT, cnT)
    q3_i = idxT_i.T.reshape(_NW, (_B * _M) // (_NW * 128), 128)
    rows_i = gat(table, q3_i)
    idxT_t, lossT_t = _encode(text, wgT, cnT)
    q3_t = idxT_t.T.reshape(_NW, (_B * _M) // (_NW * 128), 128)
    rows_t = gat(table, q3_t)
    quant_loss = (2.0 / _B) * (jnp.sum(lossT_i) + jnp.sum(lossT_t))

    similarity = _sim(rows_i.reshape(_B, _D), rows_t.reshape(_B, _D))
    return similarity, quant_loss


# speedup vs baseline: 7.3232x; 1.1149x over previous
import functools

import jax
import jax.numpy as jnp
from jax import lax
from jax.experimental import pallas as pl
from jax.experimental.pallas import tpu as pltpu
from jax.experimental.pallas import tpu_sc as plsc

_M = 32
_K = 256
_D = 512
_d = _D // _M
_B = 4096
_NB = 2 * _B

_G = 4
_MG = _M // _G
_GD = _MG * _d
_GK = _MG * _K

_RT = 1024
_RS = 512

_PREC = lax.Precision.DEFAULT

_NW = 32
_PW = (_NB * _M) // _NW
_CH = _PW // 128
_FK = 16


def _encode_body(v_ref, cent_ref, cnT_ref, idxT_ref, lossT_ref, wg_ref):
    @pl.when(pl.program_id(0) == 0)
    def _build():
        wg_ref[...] = jnp.zeros((_G, _GK, _GD), jnp.float32)
        for m in range(_M):
            g, mm = divmod(m, _MG)
            wg_ref[g, mm * _K:(mm + 1) * _K, mm * _d:(mm + 1) * _d] = (
                2.0 * cent_ref[m])

    vT = jnp.transpose(v_ref[...])
    ss = [lax.dot(wg_ref[g], vT[g * _GD:(g + 1) * _GD, :], precision=_PREC)
          for g in range(_G)]
    vsqT = vT * vT
    iota_col = lax.broadcasted_iota(jnp.int32, (_K, 1), 0).astype(jnp.float32)
    idx_rows = []
    loss_acc = None
    for m in range(_M):
        g, mm = divmod(m, _MG)
        smT = ss[g][mm * _K:(mm + 1) * _K, :]
        vnT = jnp.sum(vsqT[m * _d:(m + 1) * _d, :], axis=0, keepdims=True)
        distT = (vnT + cnT_ref[m * _K:(m + 1) * _K, :]) - smT
        minvT = jnp.min(distT, axis=0, keepdims=True)
        firstT = jnp.min(jnp.where(distT == minvT, iota_col, float(_K)),
                         axis=0, keepdims=True)
        idx_rows.append(firstT.astype(jnp.int32) + m * _K)
        loss_acc = minvT if loss_acc is None else loss_acc + minvT
    idxT_ref[...] = jnp.concatenate(idx_rows, axis=0)
    lossT_ref[...] = loss_acc


_encode = pl.pallas_call(
    _encode_body,
    grid=(_B // _RT,),
    in_specs=[
        pl.BlockSpec((_RT, _D), lambda i: (i, 0)),
        pl.BlockSpec((_M, _K, _d), lambda i: (0, 0, 0)),
        pl.BlockSpec((_M * _K, 1), lambda i: (0, 0)),
    ],
    out_specs=[
        pl.BlockSpec((_M, _RT), lambda i: (0, i)),
        pl.BlockSpec((1, _RT), lambda i: (0, i)),
    ],
    out_shape=[
        jax.ShapeDtypeStruct((_M, _B), jnp.int32),
        jax.ShapeDtypeStruct((1, _B), jnp.float32),
    ],
    scratch_shapes=[pltpu.VMEM((_G, _GK, _GD), jnp.float32)],
)


@functools.cache
def _make_gather(nrows):
    mesh = plsc.VectorSubcoreMesh(core_axis_name="c", subcore_axis_name="s")
    pw = nrows // _NW
    ch = pw // 128

    @functools.partial(
        pl.kernel,
        mesh=mesh,
        out_type=jax.ShapeDtypeStruct((nrows, _d), jnp.float32),
        scratch_types=[
            pltpu.VMEM((ch, 128), jnp.int32),
            pltpu.VMEM((_FK * 128, _d), jnp.float32),
            pltpu.SemaphoreType.DMA,
        ],
        compiler_params=pltpu.CompilerParams(use_tc_tiling_on_sc=False),
    )
    def gather(table_hbm, idx_hbm, out_hbm, idx_v, rows_v, sem):
        wid = lax.axis_index("s") * 2 + lax.axis_index("c")
        base = wid * pw
        pltpu.sync_copy(idx_hbm.at[wid], idx_v)

        def group(jj, carry):
            copies = [
                pltpu.async_copy(
                    table_hbm.at[idx_v.at[jj * _FK + t]],
                    rows_v.at[pl.ds(t * 128, 128)], sem)
                for t in range(_FK)
            ]
            for c in copies:
                c.wait()
            pltpu.sync_copy(rows_v, out_hbm.at[pl.ds(base + jj * _FK * 128, _FK * 128)])
            return carry

        lax.fori_loop(0, ch // _FK, group, 0)

    return gather


def _sim_body(img_ref, txt_ref, out_ref):
    logits = 100.0 * lax.dot_general(
        img_ref[...], txt_ref[...], (((1,), (1,)), ((), ())), precision=_PREC)
    mx = jnp.max(logits, axis=1, keepdims=True)
    e = jnp.exp(logits - mx)
    out_ref[...] = e / jnp.sum(e, axis=1, keepdims=True)


_sim = pl.pallas_call(
    _sim_body,
    grid=(_B // _RS,),
    in_specs=[
        pl.BlockSpec((_RS, _D), lambda i: (i, 0)),
        pl.BlockSpec((_B, _D), lambda i: (0, 0)),
    ],
    out_specs=pl.BlockSpec((_RS, _B), lambda i: (i, 0)),
    out_shape=jax.ShapeDtypeStruct((_B, _B), jnp.float32),
)


def kernel(image, text, centroids):
    cnT = jnp.sum(centroids ** 2, axis=2).reshape(_M * _K, 1)

    table = centroids.reshape(_M * _K, _d)
    gat = _make_gather(_B * _M)

    idxT_i, lossT_i = _encode(image, centroids, cnT)
    q3_i = idxT_i.T.reshape(_NW, (_B * _M) // (_NW * 128), 128)
    rows_i = gat(table, q3_i)
    idxT_t, lossT_t = _encode(text, centroids, cnT)
    q3_t = idxT_t.T.reshape(_NW, (_B * _M) // (_NW * 128), 128)
    rows_t = gat(table, q3_t)
    quant_loss = (2.0 / _B) * (jnp.sum(lossT_i) + jnp.sum(lossT_t))

    similarity = _sim(rows_i.reshape(_B, _D), rows_t.reshape(_B, _D))
    return similarity, quant_loss
